# Initial kernel scaffold; baseline (speedup 1.0000x reference)
import jax
import jax.numpy as jnp
from jax.experimental import pallas as pl


def kernel(x, idx_h, idx_w):
    raise NotImplementedError("write your pallas kernel here")


# speedup vs baseline: 3.4136x; 3.4136x over previous
import jax
import jax.numpy as jnp
from jax import lax
from jax.experimental import pallas as pl
from jax.experimental.pallas import tpu as pltpu
from jax.experimental.pallas import tpu_sc as plsc

B, C, H, W, S = 4, 384, 224, 224, 2
HS, WS = H // S, W // S
NPLANE = HS * WS
NIMG = H * W
NC, NS = 2, 16
NW = NC * NS
CB = C // (NW // B)
LANES = 16


CHUNK = 8 * WS


def _body(x_hbm, ih_hbm, iw_hbm, out_hbm,
          in0, in1, out_buf, comb, tmp, sem_in0, sem_in1, sem_out):
    cid = lax.axis_index("c")
    sid = lax.axis_index("s")
    wid = sid * NC + cid
    b = wid // (NW // B)
    ch0 = b * C + (wid % (NW // B)) * CB

    iota = lax.iota(jnp.int32, LANES)

    def mkcomb(t, carry):
        base = t * CHUNK
        pltpu.sync_copy(ih_hbm.at[b, pl.ds(base, CHUNK)], tmp.at[pl.ds(0, CHUNK)])
        pltpu.sync_copy(iw_hbm.at[b, pl.ds(base, CHUNK)], tmp.at[pl.ds(CHUNK, CHUNK)])

        def inner(q, carry2):
            off = q * LANES
            ihv = tmp[pl.ds(off, LANES)]
            iwv = tmp[pl.ds(CHUNK + off, LANES)]
            ev = base + off + iota
            iv = ev // WS
            jv = ev - iv * WS
            comb[pl.ds(base + off, LANES)] = (2 * iv + ihv) * W + 2 * jv + iwv
            return carry2
        lax.fori_loop(0, CHUNK // LANES, inner, 0)
        return carry
    lax.fori_loop(0, NPLANE // CHUNK, mkcomb, 0)

    bufs = (in0, in1)
    sems = (sem_in0, sem_in1)

    pltpu.async_copy(x_hbm.at[ch0 + 0], in0, sem_in0)
    pltpu.async_copy(x_hbm.at[ch0 + 1], in1, sem_in1)

    def chan(g, carry):
        for s in range(2):
            c = 2 * g + s
            buf, sem = bufs[s], sems[s]
            pltpu.make_async_copy(x_hbm.at[ch0 + c], buf, sem).wait()

            @pl.when(c > 0)
            def _wait_out():
                pltpu.make_async_copy(
                    out_buf, out_hbm.at[ch0 + c - 1], sem_out).wait()

            def gather16(k, carry2):
                for u in range(16):
                    off = k * (16 * LANES) + u * LANES
                    idx = comb[pl.ds(off, LANES)]
                    out_buf[pl.ds(off, LANES)] = plsc.load_gather(buf, [idx])
                return carry2
            lax.fori_loop(0, NPLANE // (16 * LANES), gather16, 0)

            pltpu.async_copy(out_buf, out_hbm.at[ch0 + c], sem_out)

            @pl.when(c + 2 < CB)
            def _next_in():
                pltpu.async_copy(x_hbm.at[ch0 + c + 2], buf, sem)
        return carry
    lax.fori_loop(0, CB // 2, chan, 0)

    pltpu.make_async_copy(out_buf, out_hbm.at[ch0 + CB - 1], sem_out).wait()


def kernel(x, idx_h, idx_w):
    x2 = x.reshape(B * C, NIMG)
    ihf = idx_h.reshape(B, NPLANE)
    iwf = idx_w.reshape(B, NPLANE)
    mesh = plsc.VectorSubcoreMesh(
        core_axis_name="c", subcore_axis_name="s",
        num_cores=NC, num_subcores=NS)
    f = pl.kernel(
        _body,
        out_type=jax.ShapeDtypeStruct((B * C, NPLANE), jnp.float32),
        mesh=mesh,
        compiler_params=pltpu.CompilerParams(needs_layout_passes=False),
        scratch_types=[
            pltpu.VMEM((NIMG,), jnp.float32),
            pltpu.VMEM((NIMG,), jnp.float32),
            pltpu.VMEM((NPLANE,), jnp.float32),
            pltpu.VMEM((NPLANE,), jnp.int32),
            pltpu.VMEM((2 * CHUNK,), jnp.int32),
            pltpu.SemaphoreType.DMA,
            pltpu.SemaphoreType.DMA,
            pltpu.SemaphoreType.DMA,
        ],
    )
    out2 = f(x2, ihf, iwf)
    return out2.reshape(B, C, HS, WS)


# speedup vs baseline: 3.7168x; 1.0888x over previous
import jax
import jax.numpy as jnp
from jax import lax
from jax.experimental import pallas as pl
from jax.experimental.pallas import tpu as pltpu
from jax.experimental.pallas import tpu_sc as plsc

B, C, H, W, S = 4, 384, 224, 224, 2
HS, WS = H // S, W // S
NPLANE = HS * WS
HHALF = H // 2
NHALF = NPLANE // 2
NC, NS = 2, 16
NW = NC * NS
CB = C // (NW // B)
LANES = 16
ROWCHUNK = 8
CHUNK = ROWCHUNK * WS
GUNROLL = 14


def _body(x_hbm, ih_hbm, iw_hbm, out_hbm,
          in0, in1, ob0, ob1, comb_r, comb_c, tmp,
          sem_in0, sem_in1, sem_out0, sem_out1):
    cid = lax.axis_index("c")
    sid = lax.axis_index("s")
    wid = sid * NC + cid
    b = wid // (NW // B)
    cb0 = (wid % (NW // B)) * CB

    iota = lax.iota(jnp.int32, LANES)

    def mkcomb(t, carry):
        pltpu.sync_copy(ih_hbm.at[b, pl.ds(t * ROWCHUNK, ROWCHUNK), :],
                        tmp.at[0])
        pltpu.sync_copy(iw_hbm.at[b, pl.ds(t * ROWCHUNK, ROWCHUNK), :],
                        tmp.at[1])
        for r in range(ROWCHUNK):
            i2 = 2 * (t * ROWCHUNK + r)
            for jc in range(WS // LANES):
                ihv = tmp[0, r, pl.ds(jc * LANES, LANES)]
                iwv = tmp[1, r, pl.ds(jc * LANES, LANES)]
                jv2 = 2 * (jc * LANES) + 2 * iota
                off = (t * ROWCHUNK + r) * WS + jc * LANES
                comb_r[pl.ds(off, LANES)] = i2 + ihv
                comb_c[pl.ds(off, LANES)] = jv2 + iwv
        return carry
    lax.fori_loop(0, NPLANE // CHUNK, mkcomb, 0)

    ibufs = (in0, in1)
    isems = (sem_in0, sem_in1)
    obufs = (ob0, ob1)
    osems = (sem_out0, sem_out1)

    def fire_in(item, slot):
        c = item // 2
        h = item % 2
        pltpu.async_copy(
            x_hbm.at[b, cb0 + c, pl.ds(h * HHALF, HHALF), :],
            ibufs[slot], isems[slot])

    fire_in(0, 0)
    fire_in(1, 1)

    def quad(g, carry):
        for s4 in range(4):
            item = 4 * g + s4
            c = item // 2
            h = s4 % 2
            slot = s4 % 2
            oslot = s4 // 2
            buf, isem = ibufs[slot], isems[slot]
            obuf, osem = obufs[oslot], osems[oslot]

            pltpu.make_async_copy(
                x_hbm.at[b, cb0 + c, pl.ds(h * HHALF, HHALF), :],
                buf, isem).wait()

            if h == 0:
                @pl.when(c >= 2)
                def _wait_out():
                    pltpu.make_async_copy(
                        obuf, out_hbm.at[b, cb0 + c - 2], osem).wait()

            def gather(k, carry2):
                for u in range(GUNROLL):
                    off = h * NHALF + k * (GUNROLL * LANES) + u * LANES
                    rv = comb_r[pl.ds(off, LANES)] - (h * HHALF)
                    cv = comb_c[pl.ds(off, LANES)]
                    vals = plsc.load_gather(buf, [rv, cv])
                    orow = h * (HS // 2) + 2 * k + u // (WS // LANES)
                    ocol = (u % (WS // LANES)) * LANES
                    obuf[orow, pl.ds(ocol, LANES)] = vals
                return carry2
            lax.fori_loop(0, NHALF // (GUNROLL * LANES), gather, 0)

            if h == 1:
                pltpu.async_copy(obuf, out_hbm.at[b, cb0 + c], osem)

            @pl.when(item + 2 < 2 * CB)
            def _next_in():
                fire_in(item + 2, slot)
        return carry
    lax.fori_loop(0, (2 * CB) // 4, quad, 0)

    pltpu.make_async_copy(ob0, out_hbm.at[b, cb0 + CB - 2], sem_out0).wait()
    pltpu.make_async_copy(ob1, out_hbm.at[b, cb0 + CB - 1], sem_out1).wait()


def kernel(x, idx_h, idx_w):
    mesh = plsc.VectorSubcoreMesh(
        core_axis_name="c", subcore_axis_name="s",
        num_cores=NC, num_subcores=NS)
    f = pl.kernel(
        _body,
        out_type=jax.ShapeDtypeStruct((B, C, HS, WS), jnp.float32),
        mesh=mesh,
        compiler_params=pltpu.CompilerParams(needs_layout_passes=False),
        scratch_types=[
            pltpu.VMEM((HHALF, W), jnp.float32),
            pltpu.VMEM((HHALF, W), jnp.float32),
            pltpu.VMEM((HS, WS), jnp.float32),
            pltpu.VMEM((HS, WS), jnp.float32),
            pltpu.VMEM((NPLANE,), jnp.int32),
            pltpu.VMEM((NPLANE,), jnp.int32),
            pltpu.VMEM((2, ROWCHUNK, WS), jnp.int32),
            pltpu.SemaphoreType.DMA,
            pltpu.SemaphoreType.DMA,
            pltpu.SemaphoreType.DMA,
            pltpu.SemaphoreType.DMA,
        ],
    )
    return f(x, idx_h, idx_w)


# speedup vs baseline: 40.5433x; 10.9080x over previous
import jax
import jax.numpy as jnp
from jax import lax
from jax.experimental import pallas as pl
from jax.experimental.pallas import tpu as pltpu
from jax.experimental.pallas import tpu_sc as plsc

B, C, H, W, S = 4, 384, 224, 224, 2
HS, WS = H // S, W // S
NPLANE = HS * WS
NC, NS = 2, 16
NW = NC * NS
RPT = HS // (NW // B)
LANES = 16
NCHUNK = WS // LANES


NB = 2
NR2 = 112
NCH = B * NPLANE // NW // NR2
WROWS = 24


def _body(x_hbm, ih_hbm, iw_hbm, out_hbm,
          vb0, vb1, pix, tmp, g0, g1, o0, o1):
    cid = lax.axis_index("c")
    sid = lax.axis_index("s")
    wid = sid * NC + cid
    b = wid // (NW // B)
    row0 = (wid % (NW // B)) * RPT
    p0 = b * NPLANE + row0 * WS

    iota = lax.iota(jnp.int32, LANES)

    a0 = pl.multiple_of(
        jnp.minimum((row0 // 8) * 8, HS - WROWS), 8)
    doff = row0 - a0
    pltpu.sync_copy(ih_hbm.at[b, pl.ds(a0, WROWS), :], tmp.at[0])
    pltpu.sync_copy(iw_hbm.at[b, pl.ds(a0, WROWS), :], tmp.at[1])
    base_b = b * (H * W)

    for k in range(RPT):
        i2 = 2 * (row0 + k)
        for jc in range(NCHUNK):
            ihv = tmp[0, doff + k, pl.ds(jc * LANES, LANES)]
            iwv = tmp[1, doff + k, pl.ds(jc * LANES, LANES)]
            jv2 = 2 * (jc * LANES) + 2 * iota
            pix[k, pl.ds(jc * LANES, LANES)] = (
                base_b + (i2 + ihv) * W + jv2 + iwv)

    vbs = (vb0, vb1)
    gsems = (g0, g1)
    osems = (o0, o1)

    def idx_ref(c):
        return pix.at[c]

    def fire_gather(c):
        slot = c % NB
        pltpu.async_copy(x_hbm.at[idx_ref(c)], vbs[slot], gsems[slot])

    def store_descr(c):
        slot = c % NB
        return pltpu.make_async_copy(
            vbs[slot], out_hbm.at[pl.ds(p0 + c * NR2, NR2), :], osems[slot])

    for c in range(NB):
        fire_gather(c)
    for c in range(NCH):
        slot = c % NB
        pltpu.make_async_copy(
            x_hbm.at[idx_ref(c)], vbs[slot], gsems[slot]).wait()
        pltpu.async_copy(
            vbs[slot], out_hbm.at[pl.ds(p0 + c * NR2, NR2), :], osems[slot])
        if c >= NB - 1 and c + 1 < NCH:
            store_descr(c - (NB - 1)).wait()
            fire_gather(c + 1)
    for c in range(NCH - (NB - 1), NCH):
        store_descr(c).wait()


def kernel(x, idx_h, idx_w):
    xt = jnp.transpose(x, (0, 2, 3, 1)).reshape(B * H * W, C)
    mesh = plsc.VectorSubcoreMesh(
        core_axis_name="c", subcore_axis_name="s",
        num_cores=NC, num_subcores=NS)
    f = pl.kernel(
        _body,
        out_type=jax.ShapeDtypeStruct((B * NPLANE, C), jnp.float32),
        mesh=mesh,
        compiler_params=pltpu.CompilerParams(
            needs_layout_passes=False, use_tc_tiling_on_sc=True),
        scratch_types=[
            pltpu.VMEM((NR2, C), jnp.float32),
            pltpu.VMEM((NR2, C), jnp.float32),
            pltpu.VMEM((RPT, WS), jnp.int32),
            pltpu.VMEM((2, WROWS, WS), jnp.int32),
        ] + [pltpu.SemaphoreType.DMA] * 4,
    )
    ot = f(xt, idx_h, idx_w)
    return jnp.transpose(ot.reshape(B, HS, WS, C), (0, 3, 1, 2))
